# baseline (speedup 1.0000x reference)
import functools

import jax
import jax.numpy as jnp
from jax import lax
from jax.experimental import pallas as pl
from jax.experimental.pallas import tpu as pltpu
from jax.experimental.pallas import tpu_sc as plsc

_VOCAB = 4096
_NW = 32
_ROWS_PER_W = 8192 // _NW
_C = 16
_NCH = _ROWS_PER_W // _C
_L = 16
_VB = _VOCAB // 16

_mesh = plsc.VectorSubcoreMesh(core_axis_name="c", subcore_axis_name="s")
_sc_params = pltpu.CompilerParams(needs_layout_passes=False)


@functools.partial(
    pl.kernel,
    mesh=_mesh,
    compiler_params=_sc_params,
    out_type=[
        jax.ShapeDtypeStruct((8192, _VOCAB), jnp.float32),
        jax.ShapeDtypeStruct((_NW, _L), jnp.float32),
    ],
    scratch_types=[
        pltpu.VMEM((_NCH, _C), jnp.int32),
        pltpu.VMEM((_NCH, _C), jnp.int32),
        pltpu.VMEM((_C, _VOCAB), jnp.float32),
        pltpu.VMEM((_L,), jnp.float32),
        pltpu.SemaphoreType.DMA,
    ],
)
def _sc_stream(idx_hbm, tgt_hbm, table_hbm, out_hbm, part_hbm,
               idx_v, tgt_v, rows_v, acc_v, sem):
    wid = lax.axis_index("s") * 2 + lax.axis_index("c")
    base = wid * _ROWS_PER_W
    pltpu.sync_copy(idx_hbm.at[wid], idx_v)
    pltpu.sync_copy(tgt_hbm.at[wid], tgt_v)
    row_ids = lax.iota(jnp.int32, _L)

    def body(g, tacc):
        pltpu.async_copy(table_hbm.at[idx_v.at[g]], rows_v, sem).wait()
        tvals = plsc.load_gather(rows_v, [row_ids, tgt_v[g]])
        pltpu.sync_copy(rows_v, out_hbm.at[pl.ds(base + g * _C, _C)])
        return tacc + tvals

    tacc = lax.fori_loop(0, _NCH, body, jnp.zeros((_L,), jnp.float32))
    acc_v[...] = tacc
    pltpu.sync_copy(acc_v, part_hbm.at[wid])


def _lse_count_body(idx_ref, tab_ref, s1_ref):
    i = pl.program_id(0)
    x = tab_ref[...]
    m = jnp.max(x, axis=1, keepdims=True)
    lse = jnp.log(jnp.sum(jnp.exp(x - m), axis=1, keepdims=True)) + m
    rows = i * _VB + lax.broadcasted_iota(jnp.int32, (_VB, 1), 0)

    def cbody(j, cnt):
        ids = idx_ref[:, pl.ds(j * 1024, 1024)]
        eq = (ids == rows).astype(jnp.float32)
        return cnt + jnp.sum(eq, axis=1, keepdims=True)

    cnt = lax.fori_loop(0, 8, cbody, jnp.zeros((_VB, 1), jnp.float32))
    contrib = jnp.sum(cnt * lse).reshape(1, 1)

    @pl.when(i == 0)
    def _():
        s1_ref[...] = jnp.zeros((1, 1), jnp.float32)

    s1_ref[...] += contrib


def _finalize_body(s1_ref, tpart_ref, out_ref):
    s = s1_ref[0, 0] - jnp.sum(tpart_ref[...])
    out_ref[...] = jnp.reshape(s * (1.0 / 8192.0), (1, 1))


def kernel(idx, targets, table):
    idx_c = idx.reshape(_NW, _NCH, _C)
    tgt_c = targets.reshape(_NW, _NCH, _C)
    idx_row = idx.reshape(1, 8192)

    s1 = pl.pallas_call(
        _lse_count_body,
        grid=(16,),
        in_specs=[
            pl.BlockSpec((1, 8192), lambda i: (0, 0)),
            pl.BlockSpec((_VB, _VOCAB), lambda i: (i, 0)),
        ],
        out_specs=pl.BlockSpec((1, 1), lambda i: (0, 0)),
        out_shape=jax.ShapeDtypeStruct((1, 1), jnp.float32),
    )(idx_row, table)

    logits_flat, tpart = _sc_stream(idx_c, tgt_c, table)

    loss = pl.pallas_call(
        _finalize_body,
        out_shape=jax.ShapeDtypeStruct((1, 1), jnp.float32),
    )(s1, tpart)[0, 0]

    return (logits_flat.reshape(idx.shape[0], idx.shape[1], _VOCAB), loss)
